# baseline (speedup 1.0000x reference)
import functools

import jax
import jax.numpy as jnp
from jax import lax
from jax.experimental import pallas as pl
from jax.experimental.pallas import tpu as pltpu
from jax.experimental.pallas import tpu_sc as plsc

_T, _B, _E, _H, _V = 50, 1024, 128, 256, 100000

_NW = 32
_CH = 80
_ROWS_PER_W = (_T * _B) // _NW
_NCH = _ROWS_PER_W // _CH


def _sc_embed(emb, idx3d):
    mesh = plsc.VectorSubcoreMesh(core_axis_name="c", subcore_axis_name="s")

    @functools.partial(
        pl.kernel,
        mesh=mesh,
        out_type=jax.ShapeDtypeStruct((_T * _B, _E), jnp.float32),
        scratch_types=[
            pltpu.VMEM((_NCH, _CH), jnp.int32),
            pltpu.VMEM((_CH, _E), jnp.float32),
            pltpu.SemaphoreType.DMA,
        ],
    )
    def k(emb_hbm, idx_hbm, out_hbm, idx_v, rows_v, sem):
        wid = lax.axis_index("s") * 2 + lax.axis_index("c")
        pltpu.sync_copy(idx_hbm.at[wid], idx_v)

        def body(j, carry):
            pltpu.async_copy(emb_hbm.at[idx_v.at[j]], rows_v, sem).wait()
            pltpu.sync_copy(
                rows_v, out_hbm.at[pl.ds(wid * _ROWS_PER_W + j * _CH, _CH)]
            )
            return carry

        lax.fori_loop(0, _NCH, body, 0)

    return k(emb, idx3d)


_BBLK = 256


def _lstm_body(xs_ref, wi0, wh0, b0, wi1, wh1, b1, wi2, wh2, b2,
               g_ref, bb_ref, nrm_ref, h_ref, c_ref, buf_ref):
    Bb = xs_ref.shape[1]

    def run_layer(wi, wh, b, load, store):
        wiv = wi[...].astype(jnp.bfloat16)
        whv = wh[...].astype(jnp.bfloat16)
        bv = b[...]

        def step(t, hc):
            h, c = hc
            x_t = load(t).astype(jnp.bfloat16)
            gates = (jnp.dot(x_t, wiv, preferred_element_type=jnp.float32)
                     + jnp.dot(h.astype(jnp.bfloat16), whv,
                               preferred_element_type=jnp.float32)
                     + bv)
            i = jax.nn.sigmoid(gates[:, :_H])
            f = jax.nn.sigmoid(gates[:, _H:2 * _H])
            g = jnp.tanh(gates[:, 2 * _H:3 * _H])
            o = jax.nn.sigmoid(gates[:, 3 * _H:])
            c2 = f * c + i * g
            h2 = o * jnp.tanh(c2)
            store(t, h2)
            return (h2, c2)

        z = jnp.zeros((Bb, _H), jnp.float32)
        return lax.fori_loop(0, _T, step, (z, z))

    def store_buf(t, h):
        buf_ref[t] = h

    h0, c0 = run_layer(wi0, wh0, b0, lambda t: xs_ref[t], store_buf)
    h_ref[0], c_ref[0] = h0, c0
    h1, c1 = run_layer(wi1, wh1, b1, lambda t: buf_ref[t], store_buf)
    h_ref[1], c_ref[1] = h1, c1
    h2, c2 = run_layer(wi2, wh2, b2, lambda t: buf_ref[t], lambda t, h: None)
    h_ref[2], c_ref[2] = h2, c2

    mu = jnp.mean(h2, axis=1, keepdims=True)
    d = h2 - mu
    var = jnp.mean(d * d, axis=1, keepdims=True)
    nrm_ref[...] = d * lax.rsqrt(var + 1e-5) * g_ref[...] + bb_ref[...]


def _run_lstm(xs, wi0, wh0, b0, wi1, wh1, b1, wi2, wh2, b2, ln_g2, ln_b2):
    nb = _B // _BBLK
    full = lambda i: (0, 0)
    return pl.pallas_call(
        _lstm_body,
        grid=(nb,),
        in_specs=[
            pl.BlockSpec((_T, _BBLK, _E), lambda i: (0, i, 0)),
            pl.BlockSpec((_E, 4 * _H), full), pl.BlockSpec((_H, 4 * _H), full),
            pl.BlockSpec((1, 4 * _H), full),
            pl.BlockSpec((_H, 4 * _H), full), pl.BlockSpec((_H, 4 * _H), full),
            pl.BlockSpec((1, 4 * _H), full),
            pl.BlockSpec((_H, 4 * _H), full), pl.BlockSpec((_H, 4 * _H), full),
            pl.BlockSpec((1, 4 * _H), full),
            pl.BlockSpec((1, _H), full), pl.BlockSpec((1, _H), full),
        ],
        out_specs=[
            pl.BlockSpec((_BBLK, _H), lambda i: (i, 0)),
            pl.BlockSpec((3, _BBLK, _H), lambda i: (0, i, 0)),
            pl.BlockSpec((3, _BBLK, _H), lambda i: (0, i, 0)),
        ],
        out_shape=[
            jax.ShapeDtypeStruct((_B, _H), jnp.float32),
            jax.ShapeDtypeStruct((3, _B, _H), jnp.float32),
            jax.ShapeDtypeStruct((3, _B, _H), jnp.float32),
        ],
        scratch_shapes=[pltpu.VMEM((_T, _BBLK, _H), jnp.float32)],
    )(xs, wi0, wh0, b0, wi1, wh1, b1, wi2, wh2, b2, ln_g2, ln_b2)


_VBLK = 2048


def _head_body(nrm_ref, w_ref, b_ref, out_ref):
    out_ref[...] = lax.dot_general(
        nrm_ref[...], w_ref[...], (((1,), (1,)), ((), ())),
        preferred_element_type=jnp.float32,
    ) + b_ref[...]


def _run_head(nrm, lin_W, lin_b2):
    nv = pl.cdiv(_V, _VBLK)
    return pl.pallas_call(
        _head_body,
        grid=(nv,),
        in_specs=[
            pl.BlockSpec((_B, _H), lambda i: (0, 0)),
            pl.BlockSpec((_VBLK, _H), lambda i: (i, 0)),
            pl.BlockSpec((1, _VBLK), lambda i: (0, i)),
        ],
        out_specs=pl.BlockSpec((_B, _VBLK), lambda i: (0, i)),
        out_shape=jax.ShapeDtypeStruct((_B, _V), jnp.float32),
    )(nrm, lin_W, lin_b2)


def kernel(x, emb, W_ih0, W_hh0, b_ih0, b_hh0, W_ih1, W_hh1, b_ih1, b_hh1,
           W_ih2, W_hh2, b_ih2, b_hh2, ln_g, ln_b, lin_W, lin_b):
    idx3d = x.T.reshape(_NW, _NCH, _CH)
    e = _sc_embed(emb, idx3d)
    xs = e.reshape(_T, _B, _E)

    wi0, wh0 = W_ih0.T, W_hh0.T
    wi1, wh1 = W_ih1.T, W_hh1.T
    wi2, wh2 = W_ih2.T, W_hh2.T
    b0 = (b_ih0 + b_hh0).reshape(1, 4 * _H)
    b1 = (b_ih1 + b_hh1).reshape(1, 4 * _H)
    b2 = (b_ih2 + b_hh2).reshape(1, 4 * _H)

    nrm, hs, cs = _run_lstm(xs, wi0, wh0, b0, wi1, wh1, b1, wi2, wh2, b2,
                            ln_g.reshape(1, _H), ln_b.reshape(1, _H))
    logits = _run_head(nrm, lin_W, lin_b.reshape(1, _V))
    return logits, (hs, cs)


# speedup vs baseline: 1.1273x; 1.1273x over previous
import functools

import jax
import jax.numpy as jnp
from jax import lax
from jax.experimental import pallas as pl
from jax.experimental.pallas import tpu as pltpu
from jax.experimental.pallas import tpu_sc as plsc

_T, _B, _E, _H, _V = 50, 1024, 128, 256, 100000

_NW = 32
_CH = 80
_ROWS_PER_W = (_T * _B) // _NW
_NCH = _ROWS_PER_W // _CH


def _sc_embed(emb, idx3d):
    mesh = plsc.VectorSubcoreMesh(core_axis_name="c", subcore_axis_name="s")

    @functools.partial(
        pl.kernel,
        mesh=mesh,
        out_type=jax.ShapeDtypeStruct((_T * _B, _E), jnp.float32),
        scratch_types=[
            pltpu.VMEM((_NCH, _CH), jnp.int32),
            pltpu.VMEM((_CH, _E), jnp.float32),
            pltpu.SemaphoreType.DMA,
        ],
    )
    def k(emb_hbm, idx_hbm, out_hbm, idx_v, rows_v, sem):
        wid = lax.axis_index("s") * 2 + lax.axis_index("c")
        pltpu.sync_copy(idx_hbm.at[wid], idx_v)

        def body(j, carry):
            pltpu.async_copy(emb_hbm.at[idx_v.at[j]], rows_v, sem).wait()
            pltpu.sync_copy(
                rows_v, out_hbm.at[pl.ds(wid * _ROWS_PER_W + j * _CH, _CH)]
            )
            return carry

        lax.fori_loop(0, _NCH, body, 0)

    return k(emb, idx3d)


_BBLK = 1024


def _lstm_body(xs_ref, wi0, wh0, b0, wi1, wh1, b1, wi2, wh2, b2,
               g_ref, bb_ref, nrm_ref, h_ref, c_ref, buf_ref):
    Bb = xs_ref.shape[1]

    def run_layer(wi, wh, b, load, store):
        wiv, whv, bv = wi[...], wh[...], b[...]

        def step(t, hc):
            h, c = hc
            gates = (jnp.dot(load(t), wiv, preferred_element_type=jnp.float32)
                     + jnp.dot(h.astype(jnp.bfloat16), whv,
                               preferred_element_type=jnp.float32)
                     + bv)
            t4 = jnp.tanh(gates)
            i = 0.5 * t4[:, :_H] + 0.5
            f = 0.5 * t4[:, _H:2 * _H] + 0.5
            g = t4[:, 2 * _H:3 * _H]
            o = 0.5 * t4[:, 3 * _H:] + 0.5
            c2 = f * c + i * g
            h2 = o * jnp.tanh(c2)
            store(t, h2)
            return (h2, c2)

        z = jnp.zeros((Bb, _H), jnp.float32)
        return lax.fori_loop(0, _T, step, (z, z))

    def store_buf(t, h):
        buf_ref[t] = h.astype(jnp.bfloat16)

    h0, c0 = run_layer(wi0, wh0, b0, lambda t: xs_ref[t], store_buf)
    h_ref[0], c_ref[0] = h0, c0
    h1, c1 = run_layer(wi1, wh1, b1, lambda t: buf_ref[t], store_buf)
    h_ref[1], c_ref[1] = h1, c1
    h2, c2 = run_layer(wi2, wh2, b2, lambda t: buf_ref[t], lambda t, h: None)
    h_ref[2], c_ref[2] = h2, c2

    mu = jnp.mean(h2, axis=1, keepdims=True)
    d = h2 - mu
    var = jnp.mean(d * d, axis=1, keepdims=True)
    nrm_ref[...] = d * lax.rsqrt(var + 1e-5) * g_ref[...] + bb_ref[...]


def _run_lstm(xs, wi0, wh0, b0, wi1, wh1, b1, wi2, wh2, b2, ln_g2, ln_b2):
    nb = _B // _BBLK
    full = lambda i: (0, 0)
    wspec = pl.BlockSpec((_H, 4 * _H), full)
    bspec = pl.BlockSpec((1, 4 * _H), full)
    return pl.pallas_call(
        _lstm_body,
        grid=(nb,),
        in_specs=[
            pl.BlockSpec((_T, _BBLK, _E), lambda i: (0, i, 0)),
            pl.BlockSpec((_E, 4 * _H), full, ), wspec, bspec,
            wspec, wspec, bspec,
            wspec, wspec, bspec,
            pl.BlockSpec((1, _H), full), pl.BlockSpec((1, _H), full),
        ],
        out_specs=[
            pl.BlockSpec((_BBLK, _H), lambda i: (i, 0)),
            pl.BlockSpec((3, _BBLK, _H), lambda i: (0, i, 0)),
            pl.BlockSpec((3, _BBLK, _H), lambda i: (0, i, 0)),
        ],
        out_shape=[
            jax.ShapeDtypeStruct((_B, _H), jnp.float32),
            jax.ShapeDtypeStruct((3, _B, _H), jnp.float32),
            jax.ShapeDtypeStruct((3, _B, _H), jnp.float32),
        ],
        scratch_shapes=[pltpu.VMEM((_T, _BBLK, _H), jnp.bfloat16)],
    )(xs, wi0, wh0, b0, wi1, wh1, b1, wi2, wh2, b2, ln_g2, ln_b2)


_VBLK = 2048


def _head_body(nrm_ref, w_ref, b_ref, out_ref):
    out_ref[...] = lax.dot_general(
        nrm_ref[...].astype(jnp.bfloat16), w_ref[...].astype(jnp.bfloat16),
        (((1,), (1,)), ((), ())),
        preferred_element_type=jnp.float32,
    ) + b_ref[...]


def _run_head(nrm, lin_W, lin_b2):
    nv = pl.cdiv(_V, _VBLK)
    return pl.pallas_call(
        _head_body,
        grid=(nv,),
        in_specs=[
            pl.BlockSpec((_B, _H), lambda i: (0, 0)),
            pl.BlockSpec((_VBLK, _H), lambda i: (i, 0)),
            pl.BlockSpec((1, _VBLK), lambda i: (0, i)),
        ],
        out_specs=pl.BlockSpec((_B, _VBLK), lambda i: (0, i)),
        out_shape=jax.ShapeDtypeStruct((_B, _V), jnp.float32),
    )(nrm, lin_W, lin_b2)


def _prep_gate_weights(Wih, Whh, b_ih, b_hh):
    scale = jnp.concatenate([
        jnp.full((_H,), 0.5, jnp.float32),
        jnp.full((_H,), 0.5, jnp.float32),
        jnp.ones((_H,), jnp.float32),
        jnp.full((_H,), 0.5, jnp.float32),
    ])
    wi = (Wih.T * scale).astype(jnp.bfloat16)
    wh = (Whh.T * scale).astype(jnp.bfloat16)
    b = ((b_ih + b_hh) * scale).reshape(1, 4 * _H)
    return wi, wh, b


def kernel(x, emb, W_ih0, W_hh0, b_ih0, b_hh0, W_ih1, W_hh1, b_ih1, b_hh1,
           W_ih2, W_hh2, b_ih2, b_hh2, ln_g, ln_b, lin_W, lin_b):
    idx3d = x.T.reshape(_NW, _NCH, _CH)
    e = _sc_embed(emb, idx3d)
    xs = e.reshape(_T, _B, _E).astype(jnp.bfloat16)

    wi0, wh0, b0 = _prep_gate_weights(W_ih0, W_hh0, b_ih0, b_hh0)
    wi1, wh1, b1 = _prep_gate_weights(W_ih1, W_hh1, b_ih1, b_hh1)
    wi2, wh2, b2 = _prep_gate_weights(W_ih2, W_hh2, b_ih2, b_hh2)

    nrm, hs, cs = _run_lstm(xs, wi0, wh0, b0, wi1, wh1, b1, wi2, wh2, b2,
                            ln_g.reshape(1, _H), ln_b.reshape(1, _H))
    logits = _run_head(nrm, lin_W, lin_b.reshape(1, _V))
    return logits, (hs, cs)


# speedup vs baseline: 1.1289x; 1.0015x over previous
import functools

import jax
import jax.numpy as jnp
from jax import lax
from jax.experimental import pallas as pl
from jax.experimental.pallas import tpu as pltpu
from jax.experimental.pallas import tpu_sc as plsc

_T, _B, _E, _H, _V = 50, 1024, 128, 256, 100000

_NW = 32
_CH = 80
_ROWS_PER_W = (_T * _B) // _NW
_NCH = _ROWS_PER_W // _CH


def _sc_embed(emb, idx3d):
    mesh = plsc.VectorSubcoreMesh(core_axis_name="c", subcore_axis_name="s")

    @functools.partial(
        pl.kernel,
        mesh=mesh,
        out_type=jax.ShapeDtypeStruct((_T * _B, _E), jnp.float32),
        scratch_types=[
            pltpu.VMEM((_NCH, _CH), jnp.int32),
            pltpu.VMEM((_CH, _E), jnp.float32),
            pltpu.SemaphoreType.DMA,
        ],
    )
    def k(emb_hbm, idx_hbm, out_hbm, idx_v, rows_v, sem):
        wid = lax.axis_index("s") * 2 + lax.axis_index("c")
        pltpu.sync_copy(idx_hbm.at[wid], idx_v)

        def body(j, carry):
            pltpu.async_copy(emb_hbm.at[idx_v.at[j]], rows_v, sem).wait()
            pltpu.sync_copy(
                rows_v, out_hbm.at[pl.ds(wid * _ROWS_PER_W + j * _CH, _CH)]
            )
            return carry

        lax.fori_loop(0, _NCH, body, 0)

    return k(emb, idx3d)


_BBLK = 1024


def _lstm_body(xs_ref, wi0, wh0, b0, wi1, wh1, b1, wi2, wh2, b2,
               g_ref, bb_ref, nrm_ref, h_ref, c_ref, buf_ref):
    Bb = xs_ref.shape[1]

    def run_layer(wi, wh, b, load, store):
        wiv, whv, bv = wi[...], wh[...], b[...]

        def step(t, hc):
            h, c = hc
            gates = (jnp.dot(load(t), wiv, preferred_element_type=jnp.float32)
                     + jnp.dot(h.astype(jnp.bfloat16), whv,
                               preferred_element_type=jnp.float32)
                     + bv)
            t4 = jnp.tanh(gates)
            i = 0.5 * t4[:, :_H] + 0.5
            f = 0.5 * t4[:, _H:2 * _H] + 0.5
            g = t4[:, 2 * _H:3 * _H]
            o = 0.5 * t4[:, 3 * _H:] + 0.5
            c2 = f * c + i * g
            h2 = o * jnp.tanh(c2)
            store(t, h2)
            return (h2, c2)

        z = jnp.zeros((Bb, _H), jnp.float32)
        return lax.fori_loop(0, _T, step, (z, z))

    def store_buf(t, h):
        buf_ref[t] = h.astype(jnp.bfloat16)

    h0, c0 = run_layer(wi0, wh0, b0, lambda t: xs_ref[t], store_buf)
    h_ref[0], c_ref[0] = h0, c0
    h1, c1 = run_layer(wi1, wh1, b1, lambda t: buf_ref[t], store_buf)
    h_ref[1], c_ref[1] = h1, c1
    h2, c2 = run_layer(wi2, wh2, b2, lambda t: buf_ref[t], lambda t, h: None)
    h_ref[2], c_ref[2] = h2, c2

    mu = jnp.mean(h2, axis=1, keepdims=True)
    d = h2 - mu
    var = jnp.mean(d * d, axis=1, keepdims=True)
    nrm_ref[...] = d * lax.rsqrt(var + 1e-5) * g_ref[...] + bb_ref[...]


def _run_lstm(xs, wi0, wh0, b0, wi1, wh1, b1, wi2, wh2, b2, ln_g2, ln_b2):
    nb = _B // _BBLK
    full = lambda i: (0, 0)
    wspec = pl.BlockSpec((_H, 4 * _H), full)
    bspec = pl.BlockSpec((1, 4 * _H), full)
    return pl.pallas_call(
        _lstm_body,
        grid=(nb,),
        in_specs=[
            pl.BlockSpec((_T, _BBLK, _E), lambda i: (0, i, 0)),
            pl.BlockSpec((_E, 4 * _H), full, ), wspec, bspec,
            wspec, wspec, bspec,
            wspec, wspec, bspec,
            pl.BlockSpec((1, _H), full), pl.BlockSpec((1, _H), full),
        ],
        out_specs=[
            pl.BlockSpec((_BBLK, _H), lambda i: (i, 0)),
            pl.BlockSpec((3, _BBLK, _H), lambda i: (0, i, 0)),
            pl.BlockSpec((3, _BBLK, _H), lambda i: (0, i, 0)),
        ],
        out_shape=[
            jax.ShapeDtypeStruct((_B, _H), jnp.float32),
            jax.ShapeDtypeStruct((3, _B, _H), jnp.float32),
            jax.ShapeDtypeStruct((3, _B, _H), jnp.float32),
        ],
        scratch_shapes=[pltpu.VMEM((_T, _BBLK, _H), jnp.bfloat16)],
    )(xs, wi0, wh0, b0, wi1, wh1, b1, wi2, wh2, b2, ln_g2, ln_b2)


_VBLK = 4096


def _head_body(nrm_ref, w_ref, b_ref, out_ref):
    out_ref[...] = lax.dot_general(
        nrm_ref[...].astype(jnp.bfloat16), w_ref[...].astype(jnp.bfloat16),
        (((1,), (1,)), ((), ())),
        preferred_element_type=jnp.float32,
    ) + b_ref[...]


def _run_head(nrm, lin_W, lin_b2):
    nv = pl.cdiv(_V, _VBLK)
    return pl.pallas_call(
        _head_body,
        grid=(nv,),
        in_specs=[
            pl.BlockSpec((_B, _H), lambda i: (0, 0)),
            pl.BlockSpec((_VBLK, _H), lambda i: (i, 0)),
            pl.BlockSpec((1, _VBLK), lambda i: (0, i)),
        ],
        out_specs=pl.BlockSpec((_B, _VBLK), lambda i: (0, i)),
        out_shape=jax.ShapeDtypeStruct((_B, _V), jnp.float32),
    )(nrm, lin_W, lin_b2)


def _prep_gate_weights(Wih, Whh, b_ih, b_hh):
    scale = jnp.concatenate([
        jnp.full((_H,), 0.5, jnp.float32),
        jnp.full((_H,), 0.5, jnp.float32),
        jnp.ones((_H,), jnp.float32),
        jnp.full((_H,), 0.5, jnp.float32),
    ])
    wi = (Wih.T * scale).astype(jnp.bfloat16)
    wh = (Whh.T * scale).astype(jnp.bfloat16)
    b = ((b_ih + b_hh) * scale).reshape(1, 4 * _H)
    return wi, wh, b


def kernel(x, emb, W_ih0, W_hh0, b_ih0, b_hh0, W_ih1, W_hh1, b_ih1, b_hh1,
           W_ih2, W_hh2, b_ih2, b_hh2, ln_g, ln_b, lin_W, lin_b):
    idx3d = x.T.reshape(_NW, _NCH, _CH)
    e = _sc_embed(emb, idx3d)
    xs = e.reshape(_T, _B, _E).astype(jnp.bfloat16)

    wi0, wh0, b0 = _prep_gate_weights(W_ih0, W_hh0, b_ih0, b_hh0)
    wi1, wh1, b1 = _prep_gate_weights(W_ih1, W_hh1, b_ih1, b_hh1)
    wi2, wh2, b2 = _prep_gate_weights(W_ih2, W_hh2, b_ih2, b_hh2)

    nrm, hs, cs = _run_lstm(xs, wi0, wh0, b0, wi1, wh1, b1, wi2, wh2, b2,
                            ln_g.reshape(1, _H), ln_b.reshape(1, _H))
    logits = _run_head(nrm, lin_W, lin_b.reshape(1, _V))
    return logits, (hs, cs)
